# baseline (speedup 1.0000x reference)
import functools

import jax
import jax.numpy as jnp
from jax import lax
from jax.experimental import pallas as pl
from jax.experimental.pallas import tpu as pltpu
from jax.experimental.pallas import tpu_sc as plsc

_N = 5000
_M = 100
_NP = 5120
_NT = 16
_PT = _NP // _NT
_PC = _PT // 16
_BIGI = 1 << 30
_LBL = 128


def _lanemin(x):
    s = x[0]
    for i in range(1, 16):
        s = jnp.minimum(s, x[i])
    return s


def _lanesum16(x):
    s = x[0]
    for i in range(1, 16):
        s = s + x[i]
    return s

def _srd(ref, i):
    return ref[pl.ds(i, 16)][0]

def _sc_body(amin_h, amax_h, conf_h, bmin_h, bmax_h, zero_h, out_h,
             amin_v, amax_v, conf_v, bmin_v, bmax_v,
             first_v, firstall_v, taken_v, chosen_v, chosenb_v, cc_v,
             rpart_v, partall_v, rank_v, accp_v, res_v,
             sh_first, sh_chosen, sh_part):
    w = lax.axis_index("s")
    iota16 = lax.iota(jnp.int32, 16)
    lane0 = iota16 == 0

    pltpu.sync_copy(amin_h, amin_v)
    pltpu.sync_copy(amax_h, amax_v)
    pltpu.sync_copy(conf_h, conf_v)
    pltpu.sync_copy(bmin_h, bmin_v)
    pltpu.sync_copy(bmax_h, bmax_v)
    pltpu.sync_copy(zero_h, taken_v)

    base = w * _PT

    def phase_a_label(j, _):
        b0 = _srd(bmin_v, j)
        b1 = _srd(bmax_v, j)
        blen = b1 - b0
        acc = jnp.full((16,), _BIGI, jnp.int32)
        for q in range(_PC):
            a0 = amin_v.at[pl.ds(base + q * 16, 16)][...]
            a1 = amax_v.at[pl.ds(base + q * 16, 16)][...]
            inter = jnp.maximum(jnp.minimum(a1, b1) - jnp.maximum(a0, b0), 0.0)
            union = (a1 - a0) + blen - inter
            iou = inter / union
            msk = iou > 0.5
            idx = iota16 + (base + q * 16)
            acc = jnp.minimum(acc, jnp.where(msk, idx, _BIGI))
        m = _lanemin(acc.astype(jnp.float32)).astype(jnp.int32)
        plsc.store_scatter(first_v, [jnp.full((16,), j, jnp.int32)],
                           jnp.full((16,), m, jnp.int32), mask=lane0)
        return 0

    lax.fori_loop(0, _M, phase_a_label, 0)
    pltpu.sync_copy(first_v, sh_first.at[pl.ds(w * _LBL, _LBL)])
    plsc.subcore_barrier()

    @pl.when(w == 0)
    def _phase_b():
        pltpu.sync_copy(sh_first, firstall_v)
        def red_first(j, _):
            acc = jnp.full((16,), _BIGI, jnp.int32)
            for t in range(_NT):
                acc = jnp.minimum(
                    acc, firstall_v.at[pl.ds(t * _LBL + j * 16, 16)][...])
            chosenb_v[pl.ds(j * 16, 16)] = acc
            return 0
        lax.fori_loop(0, _LBL // 16, red_first, 0)

        def phase_b_label(j, _):
            first = _srd(chosenb_v, j)
            c0 = jnp.where(first < _BIGI, lax.shift_right_logical(first, 4), 10 ** 6)
            b0 = _srd(bmin_v, j)
            b1 = _srd(bmax_v, j)
            blen = b1 - b0

            def cond(st):
                c, chosen = st
                return (c < _NP // 16) & (chosen >= _BIGI)

            def step(st):
                c, _ = st
                a0 = amin_v.at[pl.ds(c * 16, 16)][...]
                a1 = amax_v.at[pl.ds(c * 16, 16)][...]
                inter = jnp.maximum(
                    jnp.minimum(a1, b1) - jnp.maximum(a0, b0), 0.0)
                union = (a1 - a0) + blen - inter
                iou = inter / union
                tak = taken_v.at[pl.ds(c * 16, 16)][...]
                free = (iou > 0.5) & (tak == 0)
                fv = plsc.all_reduce_ffs(free)[0]
                ch = jnp.where(fv < 16, c * 16 + fv, jnp.int32(_BIGI))
                return c + 1, ch

            _, chosen = lax.while_loop(
                cond, step, (jnp.minimum(c0, _NP // 16), jnp.int32(_BIGI)))
            has = chosen < _BIGI
            m = jnp.full((16,), 1, jnp.int32)
            plsc.store_scatter(
                taken_v,
                [jnp.full((16,), jnp.minimum(chosen, _NP - 1), jnp.int32)],
                m, mask=lane0 & has)
            plsc.store_scatter(chosen_v, [jnp.full((16,), j, jnp.int32)],
                               jnp.full((16,), chosen, jnp.int32), mask=lane0)
            return 0

        for q in range(_LBL // 16):
            chosen_v[pl.ds(q * 16, 16)] = jnp.full((16,), _BIGI, jnp.int32)
        lax.fori_loop(0, _M, phase_b_label, 0)
        pltpu.sync_copy(chosen_v, sh_chosen)

    plsc.subcore_barrier()

    pltpu.sync_copy(sh_chosen, chosenb_v)
    for q in range(_LBL // 16):
        idx = chosenb_v.at[pl.ds(q * 16, 16)][...]
        vmask = idx < _BIGI
        cidx = jnp.minimum(idx, _N - 1)
        cc = plsc.load_gather(conf_v, [cidx], mask=vmask)
        cc_v[pl.ds(q * 16, 16)] = jnp.where(vmask, cc, -9.0)

    def phase_c_label(k, _):
        c = _srd(cc_v, k)
        mi = _srd(chosenb_v, k)
        acc = jnp.zeros((16,), jnp.int32)
        for q in range(_PC):
            cf = conf_v.at[pl.ds(base + q * 16, 16)][...]
            gi = iota16 + (base + q * 16)
            acc = acc + plsc.all_reduce_population_count(cf > c)
            acc = acc + plsc.all_reduce_population_count(
                (cf == c) & (gi < mi))
        r = acc[0].astype(jnp.float32)
        plsc.store_scatter(rpart_v, [jnp.full((16,), k, jnp.int32)],
                           jnp.full((16,), r, jnp.float32), mask=lane0)
        return 0

    lax.fori_loop(0, _M, phase_c_label, 0)
    pltpu.sync_copy(rpart_v, sh_part.at[pl.ds(w * _LBL, _LBL)])
    plsc.subcore_barrier()

    @pl.when(w == 0)
    def _phase_d():
        pltpu.sync_copy(sh_part, partall_v)

        def red_part(q, _):
            acc = jnp.zeros((16,), jnp.float32)
            for t in range(_NT):
                acc = acc + partall_v.at[pl.ds(t * _LBL + q * 16, 16)][...]
            rank_v[pl.ds(q * 16, 16)] = acc
            return 0
        lax.fori_loop(0, _LBL // 16, red_part, 0)

        def acc_loop(l, _):
            r_l = _srd(rank_v, l)
            v_l = jnp.where(_srd(chosenb_v, l) < _BIGI, 1.0, 0.0)
            for q in range(_LBL // 16):
                rk = rank_v.at[pl.ds(q * 16, 16)][...]
                upd = jnp.where(r_l <= rk, v_l, 0.0)
                accp_v[pl.ds(q * 16, 16)] = (
                    accp_v.at[pl.ds(q * 16, 16)][...] + upd)
            return 0

        for q in range(_LBL // 16):
            accp_v[pl.ds(q * 16, 16)] = jnp.zeros((16,), jnp.float32)
        lax.fori_loop(0, _M, acc_loop, 0)

        for q in range(_LBL // 16):
            rk = rank_v.at[pl.ds(q * 16, 16)][...]
            ak = accp_v.at[pl.ds(q * 16, 16)][...]
            accp_v[pl.ds(q * 16, 16)] = ak / (rk + 1.0)

        def suff_loop(l, carry):
            suf = carry
            r_l = _srd(rank_v, l)
            p_l = _srd(accp_v, l)
            v_l = _srd(chosenb_v, l) < _BIGI
            out = []
            for q in range(_LBL // 16):
                rk = rank_v.at[pl.ds(q * 16, 16)][...]
                s = jnp.where(v_l & (r_l >= rk), p_l, 0.0)
                out.append(jnp.maximum(suf[q], s))
            return tuple(out)

        suf0 = tuple(jnp.zeros((16,), jnp.float32) for _ in range(_LBL // 16))
        suf = lax.fori_loop(0, _M, suff_loop, suf0)

        ap = jnp.float32(0.0)
        for q in range(_LBL // 16):
            rk = rank_v.at[pl.ds(q * 16, 16)][...]
            vk = chosenb_v.at[pl.ds(q * 16, 16)][...] < _BIGI
            contrib = jnp.where(vk & (rk >= 1.0), suf[q], 0.0)
            ap = ap + _lanesum16(contrib)
        res_v[...] = jnp.full((16,), ap * (1.0 / _M), jnp.float32)
        pltpu.sync_copy(res_v, out_h)


@jax.jit
def kernel(scores, segments, gt):
    pad = _NP - _N
    amin = jnp.pad(segments[:, 0], (0, pad), constant_values=-1.0e6)
    amax = jnp.pad(segments[:, 1], (0, pad), constant_values=-1.0e6)
    conf = jnp.pad(scores, (0, pad), constant_values=-1.0)
    bmin = jnp.pad(gt[:, 0], (0, _LBL - _M), constant_values=2.0e6)
    bmax = jnp.pad(gt[:, 1], (0, _LBL - _M), constant_values=2.0e6)
    zero = jnp.zeros((_NP,), jnp.int32)

    mesh = plsc.VectorSubcoreMesh(
        core_axis_name="c", subcore_axis_name="s", num_cores=1)
    f = functools.partial(
        pl.kernel, mesh=mesh,
        out_type=jax.ShapeDtypeStruct((16,), jnp.float32),
        compiler_params=pltpu.CompilerParams(needs_layout_passes=False),
        scratch_types=[
            pltpu.VMEM((_NP,), jnp.float32),
            pltpu.VMEM((_NP,), jnp.float32),
            pltpu.VMEM((_NP,), jnp.float32),
            pltpu.VMEM((_LBL,), jnp.float32),
            pltpu.VMEM((_LBL,), jnp.float32),
            pltpu.VMEM((_LBL,), jnp.int32),
            pltpu.VMEM((_NT * _LBL,), jnp.int32),
            pltpu.VMEM((_NP,), jnp.int32),
            pltpu.VMEM((_LBL,), jnp.int32),
            pltpu.VMEM((_LBL,), jnp.int32),
            pltpu.VMEM((_LBL,), jnp.float32),
            pltpu.VMEM((_LBL,), jnp.float32),
            pltpu.VMEM((_NT * _LBL,), jnp.float32),
            pltpu.VMEM((_LBL,), jnp.float32),
            pltpu.VMEM((_LBL,), jnp.float32),
            pltpu.VMEM((16,), jnp.float32),
            pltpu.VMEM_SHARED((_NT * _LBL,), jnp.int32),
            pltpu.VMEM_SHARED((_LBL,), jnp.int32),
            pltpu.VMEM_SHARED((_NT * _LBL,), jnp.float32),
        ])(_sc_body)
    out = f(amin, amax, conf, bmin, bmax, zero)
    return out[0]


# speedup vs baseline: 1.0557x; 1.0557x over previous
import functools

import jax
import jax.numpy as jnp
from jax import lax
from jax.experimental import pallas as pl
from jax.experimental.pallas import tpu as pltpu
from jax.experimental.pallas import tpu_sc as plsc

_N = 5000
_M = 100
_NP = 5120
_NT = 16
_PT = _NP // _NT
_PC = _PT // 16
_BIGI = 1 << 30
_LBL = 128
_LC = 7


def _lanesum16(x):
    s = x[0]
    for i in range(1, 16):
        s = s + x[i]
    return s


def _sc_body(amin_h, amax_h, conf_h, bmin_h, bmax_h, zero_h, out_h,
             amin_v, amax_v, conf_v, bmin_v, bmax_v,
             first_v, firstall_v, taken_v, chosen_v, chosenb_v, cc_v,
             rpart_v, partall_v, rank_v, accp_v, tbuf_v, res_v,
             sh_first, sh_chosen, sh_part):
    w = lax.axis_index("s")
    iota16 = lax.iota(jnp.int32, 16)
    lane0 = iota16 == 0

    pltpu.sync_copy(amin_h, amin_v)
    pltpu.sync_copy(amax_h, amax_v)
    pltpu.sync_copy(conf_h, conf_v)
    pltpu.sync_copy(bmin_h, bmin_v)
    pltpu.sync_copy(bmax_h, bmax_v)

    base = w * _PT

    def phase_a_label(i, jb):
        jidx = jnp.full((16,), jb + i, jnp.int32)
        b0 = plsc.load_gather(bmin_v, [jidx])
        b1 = plsc.load_gather(bmax_v, [jidx])
        blen = b1 - b0
        acc = jnp.full((16,), _BIGI, jnp.int32)
        for q in range(_PC):
            a0 = amin_v[pl.ds(base + q * 16, 16)]
            a1 = amax_v[pl.ds(base + q * 16, 16)]
            inter = jnp.maximum(jnp.minimum(a1, b1) - jnp.maximum(a0, b0), 0.0)
            union = (a1 - a0) + blen - inter
            iou = inter / union
            idx = iota16 + (base + q * 16)
            acc = jnp.minimum(acc, jnp.where(iou > 0.5, idx, _BIGI))
        tbuf_v[pl.ds(i * 16, 16)] = acc
        return jb

    def phase_a_chunk(jc, _):
        lax.fori_loop(0, 16, phase_a_label, jc * 16)
        res = jnp.full((16,), _BIGI, jnp.int32)
        for c in range(16):
            col = plsc.load_gather(tbuf_v, [iota16 * 16 + c])
            res = jnp.minimum(res, col)
        first_v[pl.ds(jc * 16, 16)] = res
        return 0

    lax.fori_loop(0, _LC, phase_a_chunk, 0)
    pltpu.sync_copy(first_v, sh_first.at[pl.ds(w * _LBL, _LBL)])
    plsc.subcore_barrier()

    @pl.when(w == 0)
    def _phase_b():
        pltpu.sync_copy(zero_h, taken_v)
        pltpu.sync_copy(sh_first, firstall_v)

        def red_first(j, _):
            acc = jnp.full((16,), _BIGI, jnp.int32)
            for t in range(_NT):
                acc = jnp.minimum(
                    acc, firstall_v[pl.ds(t * _LBL + j * 16, 16)])
            chosenb_v[pl.ds(j * 16, 16)] = acc
            return 0
        lax.fori_loop(0, _LC, red_first, 0)

        def phase_b_label(j, _):
            jidx = jnp.full((16,), j, jnp.int32)
            first = plsc.load_gather(chosenb_v, [jidx])[0]
            c0 = jnp.where(first < _BIGI,
                           lax.shift_right_logical(first, 4), 10 ** 6)
            b0 = plsc.load_gather(bmin_v, [jidx])
            b1 = plsc.load_gather(bmax_v, [jidx])
            blen = b1 - b0

            def cond(st):
                c, chosen = st
                return (c < _NP // 16) & (chosen >= _BIGI)

            def step(st):
                c, _ = st
                a0 = amin_v[pl.ds(c * 16, 16)]
                a1 = amax_v[pl.ds(c * 16, 16)]
                inter = jnp.maximum(
                    jnp.minimum(a1, b1) - jnp.maximum(a0, b0), 0.0)
                union = (a1 - a0) + blen - inter
                iou = inter / union
                tak = taken_v[pl.ds(c * 16, 16)]
                free = (iou > 0.5) & (tak == 0)
                fv = plsc.all_reduce_ffs(free)[0]
                ch = jnp.where(fv < 16, c * 16 + fv, jnp.int32(_BIGI))
                return c + 1, ch

            _, chosen = lax.while_loop(
                cond, step, (jnp.minimum(c0, _NP // 16), jnp.int32(_BIGI)))
            has = chosen < _BIGI
            one = jnp.full((16,), 1, jnp.int32)
            plsc.store_scatter(
                taken_v,
                [jnp.full((16,), jnp.minimum(chosen, _NP - 1), jnp.int32)],
                one, mask=lane0 & has)
            plsc.store_scatter(chosen_v, [jidx],
                               jnp.full((16,), chosen, jnp.int32), mask=lane0)
            return 0

        for q in range(_LBL // 16):
            chosen_v[pl.ds(q * 16, 16)] = jnp.full((16,), _BIGI, jnp.int32)
        lax.fori_loop(0, _M, phase_b_label, 0)
        pltpu.sync_copy(chosen_v, sh_chosen)

    plsc.subcore_barrier()

    pltpu.sync_copy(sh_chosen, chosenb_v)
    for q in range(_LBL // 16):
        idx = chosenb_v[pl.ds(q * 16, 16)]
        vmask = idx < _BIGI
        cidx = jnp.minimum(idx, _N - 1)
        cc = plsc.load_gather(conf_v, [cidx], mask=vmask)
        cc_v[pl.ds(q * 16, 16)] = jnp.where(vmask, cc, -9.0)

    def phase_c_label(i, kb):
        kidx = jnp.full((16,), kb + i, jnp.int32)
        cvec = plsc.load_gather(cc_v, [kidx])
        mivec = plsc.load_gather(chosenb_v, [kidx])
        acc = jnp.zeros((16,), jnp.int32)
        for q in range(_PC):
            cf = conf_v[pl.ds(base + q * 16, 16)]
            gi = iota16 + (base + q * 16)
            acc = acc + jnp.where(cf > cvec, 1, 0)
            acc = acc + jnp.where((cf == cvec) & (gi < mivec), 1, 0)
        tbuf_v[pl.ds(i * 16, 16)] = acc
        return kb

    def phase_c_chunk(kc, _):
        lax.fori_loop(0, 16, phase_c_label, kc * 16)
        res = jnp.zeros((16,), jnp.int32)
        for c in range(16):
            res = res + plsc.load_gather(tbuf_v, [iota16 * 16 + c])
        rpart_v[pl.ds(kc * 16, 16)] = res.astype(jnp.float32)
        return 0

    lax.fori_loop(0, _LC, phase_c_chunk, 0)
    pltpu.sync_copy(rpart_v, sh_part.at[pl.ds(w * _LBL, _LBL)])
    plsc.subcore_barrier()

    @pl.when(w == 0)
    def _phase_d():
        pltpu.sync_copy(sh_part, partall_v)

        def red_part(q, _):
            acc = jnp.zeros((16,), jnp.float32)
            for t in range(_NT):
                acc = acc + partall_v[pl.ds(t * _LBL + q * 16, 16)]
            rank_v[pl.ds(q * 16, 16)] = acc
            return 0
        lax.fori_loop(0, _LC, red_part, 0)

        def acc_loop(l, _):
            lidx = jnp.full((16,), l, jnp.int32)
            r_l = plsc.load_gather(rank_v, [lidx])
            v_l = jnp.where(
                plsc.load_gather(chosenb_v, [lidx]) < _BIGI, 1.0, 0.0)
            for q in range(_LC):
                rk = rank_v[pl.ds(q * 16, 16)]
                upd = jnp.where(r_l <= rk, v_l, 0.0)
                accp_v[pl.ds(q * 16, 16)] = accp_v[pl.ds(q * 16, 16)] + upd
            return 0

        for q in range(_LC):
            accp_v[pl.ds(q * 16, 16)] = jnp.zeros((16,), jnp.float32)
        lax.fori_loop(0, _M, acc_loop, 0)

        for q in range(_LC):
            rk = rank_v[pl.ds(q * 16, 16)]
            ak = accp_v[pl.ds(q * 16, 16)]
            accp_v[pl.ds(q * 16, 16)] = ak / (rk + 1.0)

        def suff_loop(l, suf):
            lidx = jnp.full((16,), l, jnp.int32)
            r_l = plsc.load_gather(rank_v, [lidx])
            p_l = plsc.load_gather(accp_v, [lidx])
            v_l = plsc.load_gather(chosenb_v, [lidx]) < _BIGI
            out = []
            for q in range(_LC):
                rk = rank_v[pl.ds(q * 16, 16)]
                s = jnp.where(v_l & (r_l >= rk), p_l, 0.0)
                out.append(jnp.maximum(suf[q], s))
            return tuple(out)

        suf0 = tuple(jnp.zeros((16,), jnp.float32) for _ in range(_LC))
        suf = lax.fori_loop(0, _M, suff_loop, suf0)

        apv = jnp.zeros((16,), jnp.float32)
        for q in range(_LC):
            rk = rank_v[pl.ds(q * 16, 16)]
            vk = chosenb_v[pl.ds(q * 16, 16)] < _BIGI
            apv = apv + jnp.where(vk & (rk >= 1.0), suf[q], 0.0)
        ap = _lanesum16(apv)
        res_v[...] = jnp.full((16,), ap * (1.0 / _M), jnp.float32)
        pltpu.sync_copy(res_v, out_h)


@jax.jit
def kernel(scores, segments, gt):
    pad = _NP - _N
    amin = jnp.pad(segments[:, 0], (0, pad), constant_values=-1.0e6)
    amax = jnp.pad(segments[:, 1], (0, pad), constant_values=-1.0e6)
    conf = jnp.pad(scores, (0, pad), constant_values=-1.0)
    bmin = jnp.pad(gt[:, 0], (0, _LBL - _M), constant_values=2.0e6)
    bmax = jnp.pad(gt[:, 1], (0, _LBL - _M), constant_values=2.0e6)
    zero = jnp.zeros((_NP,), jnp.int32)

    mesh = plsc.VectorSubcoreMesh(
        core_axis_name="c", subcore_axis_name="s", num_cores=1)
    f = functools.partial(
        pl.kernel, mesh=mesh,
        out_type=jax.ShapeDtypeStruct((16,), jnp.float32),
        compiler_params=pltpu.CompilerParams(needs_layout_passes=False),
        scratch_types=[
            pltpu.VMEM((_NP,), jnp.float32),
            pltpu.VMEM((_NP,), jnp.float32),
            pltpu.VMEM((_NP,), jnp.float32),
            pltpu.VMEM((_LBL,), jnp.float32),
            pltpu.VMEM((_LBL,), jnp.float32),
            pltpu.VMEM((_LBL,), jnp.int32),
            pltpu.VMEM((_NT * _LBL,), jnp.int32),
            pltpu.VMEM((_NP,), jnp.int32),
            pltpu.VMEM((_LBL,), jnp.int32),
            pltpu.VMEM((_LBL,), jnp.int32),
            pltpu.VMEM((_LBL,), jnp.float32),
            pltpu.VMEM((_LBL,), jnp.float32),
            pltpu.VMEM((_NT * _LBL,), jnp.float32),
            pltpu.VMEM((_LBL,), jnp.float32),
            pltpu.VMEM((_LBL,), jnp.float32),
            pltpu.VMEM((256,), jnp.int32),
            pltpu.VMEM((16,), jnp.float32),
            pltpu.VMEM_SHARED((_NT * _LBL,), jnp.int32),
            pltpu.VMEM_SHARED((_LBL,), jnp.int32),
            pltpu.VMEM_SHARED((_NT * _LBL,), jnp.float32),
        ])(_sc_body)
    out = f(amin, amax, conf, bmin, bmax, zero)
    return out[0]
